# Initial kernel scaffold; baseline (speedup 1.0000x reference)
import jax
import jax.numpy as jnp
from jax.experimental import pallas as pl


def kernel(low_disparity, rgb, feat_w, feat_b, feat_g, feat_be, blk0_w, blk0_b, blk0_g, blk0_be, blk1_w, blk1_b, blk1_g, blk1_be, blk2_w, blk2_b, blk2_g, blk2_be, blk3_w, blk3_b, blk3_g, blk3_be, blk4_w, blk4_b, blk4_g, blk4_be, blk5_w, blk5_b, blk5_g, blk5_be, out_w, out_b):
    raise NotImplementedError("write your pallas kernel here")


# speedup vs baseline: 86.5721x; 86.5721x over previous
import functools

import jax
import jax.numpy as jnp
from jax.experimental import pallas as pl
from jax.experimental.pallas import tpu as pltpu

_LEAKY = 0.2
_EPS = 1e-5
_DILS = (1, 2, 4, 8, 1, 1)
_PAR = pltpu.CompilerParams(dimension_semantics=("parallel",))


def _pick_rows(H, dil):
    target = max(12, 2 * dil)
    cands = [r for r in range(dil, H + 1, dil) if H % r == 0]
    return min(cands, key=lambda r: abs(r - target))


def _taps(act, cin, dil, Wimg, Tp, h, bpi):
    t = pl.program_id(0)
    first = (t % bpi) == 0
    last = (t % bpi) == (bpi - 1)
    lane = jax.lax.broadcasted_iota(jnp.int32, (1, Tp), 1)
    wpos = lane % Wimg
    out = []
    for dy in (0, 1, 2):
        for dx in (0, 1, 2):
            off = dil + h + (dy - 1) * dil * Wimg + (dx - 1) * dil
            tap = act[:, off:off + Tp]
            if dx == 0:
                tap = jnp.where(wpos < dil, 0.0, tap)
            elif dx == 2:
                tap = jnp.where(wpos >= Wimg - dil, 0.0, tap)
            if dy == 0:
                tap = jnp.where(jnp.logical_and(first, lane < h), 0.0, tap)
            elif dy == 2:
                tap = jnp.where(jnp.logical_and(last, lane >= Tp - h), 0.0, tap)
            out.append(tap)
    return out


def _pad_slab(act, dil):
    cin = act.shape[0]
    z = jnp.zeros((cin, dil), jnp.float32)
    return jnp.concatenate([z, act, z], axis=1)


def _prologue(y_refs, scale_ref, shift_ref, r_refs):
    y = jnp.concatenate([r[...] for r in y_refs], axis=1).astype(jnp.float32)
    a = y * scale_ref[...] + shift_ref[...]
    a = jnp.where(a > 0, a, _LEAKY * a)
    if r_refs is not None:
        a = a + jnp.concatenate(
            [r[...] for r in r_refs], axis=1).astype(jnp.float32)
    return a


def _mxu(taps, w_ref, b_ref):
    cols = jnp.concatenate(taps, axis=0)
    return jnp.dot(w_ref[...], cols,
                   preferred_element_type=jnp.float32) + b_ref[...]


def _feat_kernel(xt, xc, xb, w, b, y_out, s, ss, *, cin, dil, Wimg, Tp, h, bpi):
    act = jnp.concatenate([xt[...], xc[...], xb[...]],
                          axis=1).astype(jnp.float32)
    taps = _taps(_pad_slab(act, dil), cin, dil, Wimg, Tp, h, bpi)
    y = _mxu(taps, w, b)
    y_out[...] = y
    s[...] = jnp.sum(y, axis=1, keepdims=True)
    ss[...] = jnp.sum(y * y, axis=1, keepdims=True)


def _block0_kernel(yt, yc, yb, sc, sh, w, b, r_out, y_out, s, ss,
                   *, cin, dil, Wimg, Tp, h, bpi):
    act = _prologue((yt, yc, yb), sc, sh, None)
    r_out[...] = act[:, h:h + Tp]
    taps = _taps(_pad_slab(act, dil), cin, dil, Wimg, Tp, h, bpi)
    y = _mxu(taps, w, b)
    y_out[...] = y
    s[...] = jnp.sum(y, axis=1, keepdims=True)
    ss[...] = jnp.sum(y * y, axis=1, keepdims=True)


def _block_kernel(yt, yc, yb, rt, rc, rb, sc, sh, w, b, r_out, y_out, s, ss,
                  *, cin, dil, Wimg, Tp, h, bpi):
    act = _prologue((yt, yc, yb), sc, sh, (rt, rc, rb))
    r_out[...] = act[:, h:h + Tp]
    taps = _taps(_pad_slab(act, dil), cin, dil, Wimg, Tp, h, bpi)
    y = _mxu(taps, w, b)
    y_out[...] = y
    s[...] = jnp.sum(y, axis=1, keepdims=True)
    ss[...] = jnp.sum(y * y, axis=1, keepdims=True)


def _out_kernel(yt, yc, yb, rt, rc, rb, sc, sh, w, b, disp, out,
                *, cin, dil, Wimg, Tp, h, bpi):
    act = _prologue((yt, yc, yb), sc, sh, (rt, rc, rb))
    taps = _taps(_pad_slab(act, dil), cin, dil, Wimg, Tp, h, bpi)
    acc = jnp.zeros((1, Tp), jnp.float32)
    for i, tap in enumerate(taps):
        acc = acc + jnp.sum(tap * w[i * cin:(i + 1) * cin, :],
                            axis=0, keepdims=True)
    out[...] = jnp.maximum(acc + b[...] + disp[...], 0.0)


def _halo_specs(C, Tp, h, R, HB):
    return [
        pl.BlockSpec((C, h), lambda t: (0, jnp.maximum(t * R - 1, 0))),
        pl.BlockSpec((C, Tp), lambda t: (0, t)),
        pl.BlockSpec((C, h), lambda t: (0, jnp.minimum((t + 1) * R, HB - 1))),
    ]


def _fold(s, ss, T, P, gamma, beta):
    C = gamma.shape[0]
    mean = (jnp.sum(s.reshape(T, C), axis=0) / P).reshape(C, 1)
    ex2 = (jnp.sum(ss.reshape(T, C), axis=0) / P).reshape(C, 1)
    var = jnp.maximum(ex2 - mean * mean, 0.0)
    scale = gamma.reshape(C, 1) * jax.lax.rsqrt(var + _EPS)
    shift = beta.reshape(C, 1) - mean * scale
    return scale, shift


def _geom(H, Wimg, P, dil):
    rows = _pick_rows(H, dil)
    Tp = rows * Wimg
    h = dil * Wimg
    return dict(Tp=Tp, h=h, T=P // Tp, R=rows // dil, HB=P // h,
                bpi=H // rows)


def kernel(low_disparity, rgb, feat_w, feat_b, feat_g, feat_be,
           blk0_w, blk0_b, blk0_g, blk0_be, blk1_w, blk1_b, blk1_g, blk1_be,
           blk2_w, blk2_b, blk2_g, blk2_be, blk3_w, blk3_b, blk3_g, blk3_be,
           blk4_w, blk4_b, blk4_g, blk4_be, blk5_w, blk5_b, blk5_g, blk5_be,
           out_w, out_b):
    N, _, w_low = low_disparity.shape
    _, _, H, Wimg = rgb.shape
    P = N * H * Wimg
    C = feat_w.shape[0]
    cin0 = feat_w.shape[1]

    twice = jax.image.resize(low_disparity[:, None].astype(jnp.float32),
                             (N, 1, H, Wimg), method="bilinear")
    if Wimg / w_low >= 1.5:
        twice = twice * 8.0
    x = jnp.concatenate([twice, rgb.astype(jnp.float32)], axis=1)
    a4 = jnp.transpose(x, (1, 0, 2, 3)).reshape(cin0, P)
    disp = jnp.transpose(twice, (1, 0, 2, 3)).reshape(1, P)

    def wflat(w):
        co, ci = w.shape[0], w.shape[1]
        return jnp.transpose(w, (0, 2, 3, 1)).reshape(co, 9 * ci).astype(
            jnp.float32)

    blks = ((blk0_w, blk0_b, blk0_g, blk0_be), (blk1_w, blk1_b, blk1_g, blk1_be),
            (blk2_w, blk2_b, blk2_g, blk2_be), (blk3_w, blk3_b, blk3_g, blk3_be),
            (blk4_w, blk4_b, blk4_g, blk4_be), (blk5_w, blk5_b, blk5_g, blk5_be))

    g = _geom(H, Wimg, P, 1)
    T = g["T"]
    yf, s, ss = pl.pallas_call(
        functools.partial(_feat_kernel, cin=cin0, dil=1, Wimg=Wimg,
                          Tp=g["Tp"], h=g["h"], bpi=g["bpi"]),
        out_shape=(jax.ShapeDtypeStruct((C, P), jnp.float32),
                   jax.ShapeDtypeStruct((T * C, 1), jnp.float32),
                   jax.ShapeDtypeStruct((T * C, 1), jnp.float32)),
        grid=(T,),
        in_specs=_halo_specs(cin0, g["Tp"], g["h"], g["R"], g["HB"]) + [
            pl.BlockSpec((C, 9 * cin0), lambda t: (0, 0)),
            pl.BlockSpec((C, 1), lambda t: (0, 0)),
        ],
        out_specs=(pl.BlockSpec((C, g["Tp"]), lambda t: (0, t)),
                   pl.BlockSpec((C, 1), lambda t: (t, 0)),
                   pl.BlockSpec((C, 1), lambda t: (t, 0))),
        compiler_params=_PAR,
    )(a4, a4, a4, wflat(feat_w), feat_b.reshape(C, 1).astype(jnp.float32))
    scale, shift = _fold(s, ss, T, P, feat_g, feat_be)

    y_prev, r_prev = yf, None
    for i, dil in enumerate(_DILS):
        w_i, b_i, g_i, be_i = blks[i]
        g = _geom(H, Wimg, P, dil)
        T = g["T"]
        hs = functools.partial(_halo_specs, Tp=g["Tp"], h=g["h"], R=g["R"],
                               HB=g["HB"])
        common = [
            pl.BlockSpec((C, 1), lambda t: (0, 0)),
            pl.BlockSpec((C, 1), lambda t: (0, 0)),
            pl.BlockSpec((C, 9 * C), lambda t: (0, 0)),
            pl.BlockSpec((C, 1), lambda t: (0, 0)),
        ]
        if i == 0:
            kern = functools.partial(_block0_kernel, cin=C, dil=dil, Wimg=Wimg,
                                     Tp=g["Tp"], h=g["h"], bpi=g["bpi"])
            in_specs = hs(C) + common
            operands = (y_prev, y_prev, y_prev)
        else:
            kern = functools.partial(_block_kernel, cin=C, dil=dil, Wimg=Wimg,
                                     Tp=g["Tp"], h=g["h"], bpi=g["bpi"])
            in_specs = hs(C) + hs(C) + common
            operands = (y_prev, y_prev, y_prev, r_prev, r_prev, r_prev)
        r_i, y_i, s, ss = pl.pallas_call(
            kern,
            out_shape=(jax.ShapeDtypeStruct((C, P), jnp.float32),
                       jax.ShapeDtypeStruct((C, P), jnp.float32),
                       jax.ShapeDtypeStruct((T * C, 1), jnp.float32),
                       jax.ShapeDtypeStruct((T * C, 1), jnp.float32)),
            grid=(T,),
            in_specs=in_specs,
            out_specs=(pl.BlockSpec((C, g["Tp"]), lambda t: (0, t)),
                       pl.BlockSpec((C, g["Tp"]), lambda t: (0, t)),
                       pl.BlockSpec((C, 1), lambda t: (t, 0)),
                       pl.BlockSpec((C, 1), lambda t: (t, 0))),
            compiler_params=_PAR,
        )(*operands, scale, shift, wflat(w_i),
          b_i.reshape(C, 1).astype(jnp.float32))
        scale, shift = _fold(s, ss, T, P, g_i, be_i)
        y_prev, r_prev = y_i, r_i

    g = _geom(H, Wimg, P, 1)
    T = g["T"]
    hs = _halo_specs(C, g["Tp"], g["h"], g["R"], g["HB"])
    w_col = jnp.transpose(out_w, (2, 3, 1, 0)).reshape(9 * C, 1).astype(
        jnp.float32)
    out_flat = pl.pallas_call(
        functools.partial(_out_kernel, cin=C, dil=1, Wimg=Wimg,
                          Tp=g["Tp"], h=g["h"], bpi=g["bpi"]),
        out_shape=jax.ShapeDtypeStruct((1, P), jnp.float32),
        grid=(T,),
        in_specs=hs + [
            pl.BlockSpec((C, g["h"]),
                         lambda t, R=g["R"]: (0, jnp.maximum(t * R - 1, 0))),
            pl.BlockSpec((C, g["Tp"]), lambda t: (0, t)),
            pl.BlockSpec((C, g["h"]),
                         lambda t, R=g["R"], HB=g["HB"]:
                         (0, jnp.minimum((t + 1) * R, HB - 1))),
            pl.BlockSpec((C, 1), lambda t: (0, 0)),
            pl.BlockSpec((C, 1), lambda t: (0, 0)),
            pl.BlockSpec((9 * C, 1), lambda t: (0, 0)),
            pl.BlockSpec((1, 1), lambda t: (0, 0)),
            pl.BlockSpec((1, g["Tp"]), lambda t: (0, t)),
        ],
        out_specs=pl.BlockSpec((1, g["Tp"]), lambda t: (0, t)),
        compiler_params=_PAR,
    )(y_prev, y_prev, y_prev, r_prev, r_prev, r_prev, scale, shift,
      w_col, out_b.reshape(1, 1).astype(jnp.float32), disp)
    return out_flat.reshape(N, H, Wimg)
